# baseline (speedup 1.0000x reference)
import functools

import jax
import jax.numpy as jnp
from jax import lax
from jax.experimental import pallas as pl
from jax.experimental.pallas import tpu as pltpu
from jax.experimental.pallas import tpu_sc as plsc

S, Q, E, D0, D1, D2 = 20, 500, 16000, 128, 128, 128
NP = 512
QUARTER = 128
NWORKER = 32
CHUNK = 3200
NCHUNK = E // CHUNK
GROUPS = CHUNK // 16
HALF = S // 2

_sc_mesh = plsc.VectorSubcoreMesh(core_axis_name="c", subcore_axis_name="s")


def _make_build_adjacency(ns):
    ntask = ns * (NP // QUARTER)

    @functools.partial(
        pl.kernel,
        out_type=jax.ShapeDtypeStruct((ns, NP, NP), jnp.float32),
        mesh=_sc_mesh,
        scratch_types=[
            pltpu.VMEM((2, 2, CHUNK), jnp.int32),
            pltpu.VMEM((QUARTER, NP), jnp.float32),
            pltpu.SemaphoreType.DMA,
            pltpu.SemaphoreType.DMA,
        ],
        compiler_params=pltpu.CompilerParams(
            needs_layout_passes=False, use_tc_tiling_on_sc=True),
    )
    def build(sm_hbm, a_hbm, ebuf_v, acc_v, sem0, sem1):
        wid = lax.axis_index("s") * 2 + lax.axis_index("c")
        zeros16 = jnp.zeros((16,), jnp.float32)
        ones16 = jnp.ones((16,), jnp.float32)
        sems = (sem0, sem1)
        UNROLL = 16

        def start_chunk(sl, c, buf):
            return pltpu.async_copy(
                sm_hbm.at[sl, :, pl.ds(c * CHUNK, CHUNK)],
                ebuf_v.at[buf], sems[buf])

        for rnd in range((ntask + NWORKER - 1) // NWORKER):
            task = rnd * NWORKER + wid

            @pl.when(task < ntask)
            def _():
                sl = task % ns
                base = (task // ns) * QUARTER

                copies = [None] * NCHUNK
                copies[0] = start_chunk(sl, 0, 0)

                @plsc.parallel_loop(0, QUARTER, unroll=4)
                def _(i):
                    for j in range(NP // 16):
                        acc_v[i, pl.ds(j * 16, 16)] = zeros16

                for c in range(NCHUNK):
                    cur = c % 2
                    if c + 1 < NCHUNK:
                        copies[c + 1] = start_chunk(sl, c + 1, 1 - cur)
                    copies[c].wait()

                    @plsc.parallel_loop(0, GROUPS, unroll=UNROLL)
                    def _(g):
                        off = g * 16
                        r16 = ebuf_v[cur, 0, pl.ds(off, 16)]
                        c16 = ebuf_v[cur, 1, pl.ds(off, 16)]
                        local = c16 - base
                        mask = local.astype(jnp.uint32) < QUARTER
                        plsc.addupdate_scatter(
                            acc_v, [local, r16], ones16, mask=mask)

                pltpu.sync_copy(acc_v, a_hbm.at[sl, pl.ds(base, QUARTER), :])

    return build


def _split(x):
    hi = x.astype(jnp.bfloat16)
    lo = (x - hi.astype(jnp.float32)).astype(jnp.bfloat16)
    return hi, lo


def _mm3(xs, ys):
    xh, xl = xs
    yh, yl = ys
    def d(p, q):
        return jnp.dot(p, q, preferred_element_type=jnp.float32)
    return d(xh, yh) + d(xh, yl) + d(xl, yh)


def _amm(ah, ys):
    yh, yl = ys
    def d(p, q):
        return jnp.dot(p, q, preferred_element_type=jnp.float32)
    return d(ah, yh) + d(ah, yl)


TCB = 4


def _gcn_body(a_ref, qe_ref, w1_ref, w2_ref, b1_ref, b2_ref, out_ref, h0_ref):
    @pl.when(pl.program_id(0) == 0)
    def _():
        h0_ref[pl.ds(Q, NP - Q), :] = jnp.zeros((NP - Q, D1), jnp.float32)
        h0_ref[pl.ds(0, Q), :] = _mm3(_split(qe_ref[...]), _split(w1_ref[...]))

    w2s = _split(w2_ref[...])
    for k in range(TCB):
        a = a_ref[k]
        deg = jnp.sum(a, axis=1) + 1.0
        dinv = lax.rsqrt(deg)[:, None]
        ah = a.astype(jnp.bfloat16)

        xs1 = dinv * h0_ref[...]
        x1 = jnp.maximum(
            dinv * (_amm(ah, _split(xs1)) + xs1) + b1_ref[...], 0.0)
        xs2 = dinv * _mm3(_split(x1), w2s)
        x2 = jnp.maximum(
            dinv * (_amm(ah, _split(xs2)) + xs2) + b2_ref[...], 0.0)
        out_ref[k] = x2[:Q]


def _gcn_tc(a, qe, w1, w2, b1, b2):
    ns = a.shape[0]
    return pl.pallas_call(
        _gcn_body,
        grid=(ns // TCB,),
        in_specs=[
            pl.BlockSpec((TCB, NP, NP), lambda s: (s, 0, 0)),
            pl.BlockSpec((Q, D0), lambda s: (0, 0)),
            pl.BlockSpec((D0, D1), lambda s: (0, 0)),
            pl.BlockSpec((D1, D2), lambda s: (0, 0)),
            pl.BlockSpec((D1,), lambda s: (0,)),
            pl.BlockSpec((D2,), lambda s: (0,)),
        ],
        out_specs=pl.BlockSpec((TCB, Q, D2), lambda s: (s, 0, 0)),
        out_shape=jax.ShapeDtypeStruct((ns, Q, D2), jnp.float32),
        scratch_shapes=[pltpu.VMEM((NP, D1), jnp.float32)],
    )(a, qe, w1, w2, b1, b2)


_build_full = _make_build_adjacency(S)


def kernel(slice_matrices, qubit_embs, W1, b1, W2, b2):
    sm = slice_matrices.astype(jnp.int32)
    a = _build_full(sm)
    out = _gcn_tc(a, qubit_embs, W1, W2, b1, b2)
    return out.reshape(S * Q, D2)


# speedup vs baseline: 1.0404x; 1.0404x over previous
import functools

import jax
import jax.numpy as jnp
from jax import lax
from jax.experimental import pallas as pl
from jax.experimental.pallas import tpu as pltpu
from jax.experimental.pallas import tpu_sc as plsc

S, Q, E, D0, D1, D2 = 20, 500, 16000, 128, 128, 128
NP = 512
QUARTER = 128
NWORKER = 32
CHUNK = 3200
NCHUNK = E // CHUNK
GROUPS = CHUNK // 16
HALF = S // 2

_sc_mesh = plsc.VectorSubcoreMesh(core_axis_name="c", subcore_axis_name="s")


def _make_build_adjacency(ns):
    ntask = ns * (NP // QUARTER)

    @functools.partial(
        pl.kernel,
        out_type=jax.ShapeDtypeStruct((ns, NP, NP), jnp.float32),
        mesh=_sc_mesh,
        scratch_types=[
            pltpu.VMEM((2, 2, CHUNK), jnp.int32),
            pltpu.VMEM((QUARTER, NP), jnp.float32),
            pltpu.SemaphoreType.DMA,
            pltpu.SemaphoreType.DMA,
        ],
        compiler_params=pltpu.CompilerParams(
            needs_layout_passes=False, use_tc_tiling_on_sc=True),
    )
    def build(sm_hbm, a_hbm, ebuf_v, acc_v, sem0, sem1):
        wid = lax.axis_index("s") * 2 + lax.axis_index("c")
        zeros16 = jnp.zeros((16,), jnp.float32)
        ones16 = jnp.ones((16,), jnp.float32)
        sems = (sem0, sem1)
        UNROLL = 10

        def start_chunk(sl, c, buf):
            return pltpu.async_copy(
                sm_hbm.at[sl, :, pl.ds(c * CHUNK, CHUNK)],
                ebuf_v.at[buf], sems[buf])

        for rnd in range((ntask + NWORKER - 1) // NWORKER):
            task = rnd * NWORKER + wid

            @pl.when(task < ntask)
            def _():
                sl = task % ns
                base = (task // ns) * QUARTER

                copies = [None] * NCHUNK
                copies[0] = start_chunk(sl, 0, 0)

                @plsc.parallel_loop(0, QUARTER, unroll=2)
                def _(i):
                    for j in range(NP // 16):
                        acc_v[i, pl.ds(j * 16, 16)] = zeros16

                for c in range(NCHUNK):
                    cur = c % 2
                    if c + 1 < NCHUNK:
                        copies[c + 1] = start_chunk(sl, c + 1, 1 - cur)
                    copies[c].wait()

                    @plsc.parallel_loop(0, GROUPS, unroll=UNROLL)
                    def _(g):
                        off = g * 16
                        r16 = ebuf_v[cur, 0, pl.ds(off, 16)]
                        c16 = ebuf_v[cur, 1, pl.ds(off, 16)]
                        local = c16 - base
                        mask = local.astype(jnp.uint32) < QUARTER
                        plsc.addupdate_scatter(
                            acc_v, [local, r16], ones16, mask=mask)

                pltpu.sync_copy(acc_v, a_hbm.at[sl, pl.ds(base, QUARTER), :])

    return build


def _split(x):
    hi = x.astype(jnp.bfloat16)
    lo = (x - hi.astype(jnp.float32)).astype(jnp.bfloat16)
    return hi, lo


def _mm3(xs, ys):
    xh, xl = xs
    yh, yl = ys
    def d(p, q):
        return jnp.dot(p, q, preferred_element_type=jnp.float32)
    return d(xh, yh) + d(xh, yl) + d(xl, yh)


def _amm(ah, ys):
    yh, yl = ys
    def d(p, q):
        return jnp.dot(p, q, preferred_element_type=jnp.float32)
    return d(ah, yh) + d(ah, yl)


TCB = 4


def _gcn_body(a_ref, qe_ref, w1_ref, w2_ref, b1_ref, b2_ref, out_ref, h0_ref):
    @pl.when(pl.program_id(0) == 0)
    def _():
        h0_ref[pl.ds(Q, NP - Q), :] = jnp.zeros((NP - Q, D1), jnp.float32)
        h0_ref[pl.ds(0, Q), :] = _mm3(_split(qe_ref[...]), _split(w1_ref[...]))

    w2s = _split(w2_ref[...])
    for k in range(TCB):
        a = a_ref[k]
        deg = jnp.sum(a, axis=1) + 1.0
        dinv = lax.rsqrt(deg)[:, None]
        ah = a.astype(jnp.bfloat16)

        xs1 = dinv * h0_ref[...]
        x1 = jnp.maximum(
            dinv * (_amm(ah, _split(xs1)) + xs1) + b1_ref[...], 0.0)
        xs2 = dinv * _mm3(_split(x1), w2s)
        x2 = jnp.maximum(
            dinv * (_amm(ah, _split(xs2)) + xs2) + b2_ref[...], 0.0)
        out_ref[k] = x2[:Q]


def _gcn_tc(a, qe, w1, w2, b1, b2):
    ns = a.shape[0]
    return pl.pallas_call(
        _gcn_body,
        grid=(ns // TCB,),
        in_specs=[
            pl.BlockSpec((TCB, NP, NP), lambda s: (s, 0, 0)),
            pl.BlockSpec((Q, D0), lambda s: (0, 0)),
            pl.BlockSpec((D0, D1), lambda s: (0, 0)),
            pl.BlockSpec((D1, D2), lambda s: (0, 0)),
            pl.BlockSpec((D1,), lambda s: (0,)),
            pl.BlockSpec((D2,), lambda s: (0,)),
        ],
        out_specs=pl.BlockSpec((TCB, Q, D2), lambda s: (s, 0, 0)),
        out_shape=jax.ShapeDtypeStruct((ns, Q, D2), jnp.float32),
        scratch_shapes=[pltpu.VMEM((NP, D1), jnp.float32)],
    )(a, qe, w1, w2, b1, b2)


_build_full = _make_build_adjacency(S)


def kernel(slice_matrices, qubit_embs, W1, b1, W2, b2):
    sm = slice_matrices.astype(jnp.int32)
    a = _build_full(sm)
    out = _gcn_tc(a, qubit_embs, W1, W2, b1, b2)
    return out.reshape(S * Q, D2)
